# baseline (speedup 1.0000x reference)
import jax
import jax.numpy as jnp
from jax.experimental import pallas as pl
from jax.experimental.pallas import tpu as pltpu

_LANES = 512
_GRID = 2
_STREAMS = 2


def _l1_mean_kernel(inv_n_ref, *refs):
    *in_refs, o_ref, acc_ref = refs
    i = pl.program_id(0)

    @pl.when(i == 0)
    def _init():
        acc_ref[...] = jnp.zeros_like(acc_ref)

    n = len(in_refs) // 2
    part = jnp.zeros_like(acc_ref[...])
    for s in range(n):
        part += jnp.sum(jnp.abs(in_refs[s][...] - in_refs[n + s][...]),
                        axis=0, keepdims=True)
    acc_ref[...] += part

    @pl.when(i == pl.num_programs(0) - 1)
    def _finish():
        o_ref[0, 0] = jnp.sum(acc_ref[...]) * inv_n_ref[0]


def kernel(pred_logits, gt_sdf):
    p = pred_logits.reshape(-1, _LANES)
    g = gt_sdf.reshape(-1, _LANES)
    rows = p.shape[0]
    blk = rows // (_STREAMS * _GRID)
    inv_n = jnp.full((1,), 1.0 / p.size, dtype=jnp.float32)
    specs = [
        pl.BlockSpec((blk, _LANES), lambda i, s=s: (i + s * _GRID, 0))
        for s in range(_STREAMS)
    ]
    total = pl.pallas_call(
        _l1_mean_kernel,
        grid=(_GRID,),
        in_specs=[pl.BlockSpec(memory_space=pltpu.SMEM)] + specs + specs,
        out_specs=pl.BlockSpec(memory_space=pltpu.SMEM),
        out_shape=jax.ShapeDtypeStruct((1, 1), jnp.float32),
        scratch_shapes=[pltpu.VMEM((1, _LANES), jnp.float32)],
    )(inv_n, *([p] * _STREAMS), *([g] * _STREAMS))
    return total[0, 0]


# speedup vs baseline: 1.0220x; 1.0220x over previous
import jax
import jax.numpy as jnp
from jax.experimental import pallas as pl
from jax.experimental.pallas import tpu as pltpu

_LANES = 512
_GRID = 8
_STREAMS = 2


def _l1_mean_kernel(inv_n_ref, *refs):
    *in_refs, o_ref, acc_ref = refs
    i = pl.program_id(0)

    @pl.when(i == 0)
    def _init():
        acc_ref[...] = jnp.zeros_like(acc_ref)

    n = len(in_refs) // 2
    part = jnp.zeros_like(acc_ref[...])
    for s in range(n):
        part += jnp.sum(jnp.abs(in_refs[s][...] - in_refs[n + s][...]),
                        axis=0, keepdims=True)
    acc_ref[...] += part

    @pl.when(i == pl.num_programs(0) - 1)
    def _finish():
        o_ref[0, 0] = jnp.sum(acc_ref[...]) * inv_n_ref[0]


def kernel(pred_logits, gt_sdf):
    p = pred_logits.reshape(-1, _LANES)
    g = gt_sdf.reshape(-1, _LANES)
    rows = p.shape[0]
    blk = rows // (_STREAMS * _GRID)
    inv_n = jnp.full((1,), 1.0 / p.size, dtype=jnp.float32)
    specs = [
        pl.BlockSpec((blk, _LANES), lambda i, s=s: (i + s * _GRID, 0))
        for s in range(_STREAMS)
    ]
    total = pl.pallas_call(
        _l1_mean_kernel,
        grid=(_GRID,),
        in_specs=[pl.BlockSpec(memory_space=pltpu.SMEM)] + specs + specs,
        out_specs=pl.BlockSpec(memory_space=pltpu.SMEM),
        out_shape=jax.ShapeDtypeStruct((1, 1), jnp.float32),
        scratch_shapes=[pltpu.VMEM((1, _LANES), jnp.float32)],
    )(inv_n, *([p] * _STREAMS), *([g] * _STREAMS))
    return total[0, 0]


# speedup vs baseline: 1.0309x; 1.0086x over previous
import jax
import jax.numpy as jnp
from jax.experimental import pallas as pl
from jax.experimental.pallas import tpu as pltpu

_LANES = 512
_GRID = 4
_STREAMS = 2


def _l1_mean_kernel(inv_n_ref, *refs):
    *in_refs, o_ref, acc_ref = refs
    i = pl.program_id(0)

    @pl.when(i == 0)
    def _init():
        acc_ref[...] = jnp.zeros_like(acc_ref)

    n = len(in_refs) // 2
    part = jnp.zeros_like(acc_ref[...])
    for s in range(n):
        part += jnp.sum(jnp.abs(in_refs[s][...] - in_refs[n + s][...]),
                        axis=0, keepdims=True)
    acc_ref[...] += part

    @pl.when(i == pl.num_programs(0) - 1)
    def _finish():
        o_ref[0, 0] = jnp.sum(acc_ref[...]) * inv_n_ref[0]


def kernel(pred_logits, gt_sdf):
    p = pred_logits.reshape(-1, _LANES)
    g = gt_sdf.reshape(-1, _LANES)
    rows = p.shape[0]
    blk = rows // (_STREAMS * _GRID)
    inv_n = jnp.full((1,), 1.0 / p.size, dtype=jnp.float32)
    specs = [
        pl.BlockSpec((blk, _LANES), lambda i, s=s: (2 * i + s, 0))
        for s in range(_STREAMS)
    ]
    total = pl.pallas_call(
        _l1_mean_kernel,
        grid=(_GRID,),
        in_specs=[pl.BlockSpec(memory_space=pltpu.SMEM)] + specs + specs,
        out_specs=pl.BlockSpec(memory_space=pltpu.SMEM),
        out_shape=jax.ShapeDtypeStruct((1, 1), jnp.float32),
        scratch_shapes=[pltpu.VMEM((1, _LANES), jnp.float32)],
    )(inv_n, *([p] * _STREAMS), *([g] * _STREAMS))
    return total[0, 0]
